# baseline (speedup 1.0000x reference)
import functools
import jax
import jax.numpy as jnp
from jax import lax
from jax.experimental import pallas as pl
from jax.experimental.pallas import tpu as pltpu
from jax.experimental.pallas import tpu_sc as plsc

INPUT_MIN, INPUT_MAX = 0.0, 1.0

_NC, _NS = 2, 16
_NW = _NC * _NS
_B, _IN, _OUT, _K = 1024, 128, 128, 16
_RPW = _B // _NW


def _sc_body(x_hbm, w2_hbm, bias_hbm, out_hbm, xv, idxv, rows, outv, biasv, sem):
    wid = lax.axis_index("s") * _NC + lax.axis_index("c")
    base = wid * _RPW
    pltpu.sync_copy(x_hbm.at[pl.ds(base, _RPW)], xv)
    pltpu.sync_copy(bias_hbm, biasv)
    lane = lax.broadcasted_iota(jnp.int32, (16,), 0)

    def row_body(r, carry):
        for j in range(_IN // 16):
            xs = xv[r, pl.ds(16 * j, 16)]
            xn = (xs - INPUT_MIN) / (INPUT_MAX - INPUT_MIN)
            ki = jnp.clip((xn * _K).astype(jnp.int32), 0, _K - 1)
            idxv[pl.ds(16 * j, 16)] = (lane + 16 * j) * _K + ki
        pltpu.async_copy(w2_hbm.at[idxv], rows, sem).wait()

        def group_body(jj, acc):
            xs = xv[r, pl.ds(jj * 16, 16)]
            acc = list(acc)
            for l in range(16):
                i = jj * 16 + l
                xb = jnp.full((16,), xs[l], jnp.float32)
                for g in range(_OUT // 16):
                    acc[g] = (acc[g] + xb * rows[i, pl.ds(16 * g, 16)]
                              + rows[i, pl.ds(_OUT + 16 * g, 16)])
            return tuple(acc)

        acc0 = tuple(jnp.zeros((16,), jnp.float32) for _ in range(_OUT // 16))
        acc = lax.fori_loop(0, _IN // 16, group_body, acc0)
        for g in range(_OUT // 16):
            outv[r, pl.ds(16 * g, 16)] = acc[g] + biasv[pl.ds(16 * g, 16)]
        return carry

    lax.fori_loop(0, _RPW, row_body, 0)
    pltpu.sync_copy(outv, out_hbm.at[pl.ds(base, _RPW)])


def kernel(x, slopes, intercepts, bias):
    out_dim, in_dim, k = slopes.shape
    b = x.shape[0]
    s_r = jnp.transpose(slopes, (1, 2, 0)).reshape(in_dim * k, out_dim)
    t_r = jnp.transpose(intercepts, (1, 2, 0)).reshape(in_dim * k, out_dim)
    w2 = jnp.concatenate([s_r, t_r], axis=1)

    mesh = plsc.VectorSubcoreMesh(core_axis_name="c", subcore_axis_name="s")
    run = functools.partial(
        pl.kernel,
        mesh=mesh,
        out_type=jax.ShapeDtypeStruct((b, out_dim), jnp.float32),
        scratch_types=[
            pltpu.VMEM((_RPW, in_dim), jnp.float32),
            pltpu.VMEM((in_dim,), jnp.int32),
            pltpu.VMEM((in_dim, 2 * out_dim), jnp.float32),
            pltpu.VMEM((_RPW, out_dim), jnp.float32),
            pltpu.VMEM((out_dim,), jnp.float32),
            pltpu.SemaphoreType.DMA,
        ],
    )(_sc_body)
    return run(x, w2, bias)


# speedup vs baseline: 17.1675x; 17.1675x over previous
import jax
import jax.numpy as jnp
from jax.experimental import pallas as pl
from jax.experimental.pallas import tpu as pltpu

INPUT_MIN, INPUT_MAX = 0.0, 1.0


def _spline_body(x_ref, w_hbm, bias_ref, out_ref, wv_ref, sems):
    num_k = wv_ref.shape[0]
    in_dim = x_ref.shape[1]
    copies = [
        pltpu.make_async_copy(w_hbm.at[kk], wv_ref.at[kk], sems.at[kk])
        for kk in range(num_k)
    ]
    for c in copies:
        c.start()
    xv = x_ref[:]
    x_norm = (xv - INPUT_MIN) / (INPUT_MAX - INPUT_MIN)
    idx = jnp.clip(jnp.floor(x_norm * num_k), 0.0, num_k - 1.0).astype(jnp.bfloat16)
    xbf = xv.astype(jnp.bfloat16)
    acc = jnp.zeros((xv.shape[0], wv_ref.shape[2]), jnp.float32)
    for kk in range(num_k):
        sel = idx == jnp.bfloat16(kk)
        xm = jnp.where(sel, xbf, jnp.bfloat16(0))
        mask = jnp.where(sel, jnp.bfloat16(1), jnp.bfloat16(0))
        copies[kk].wait()
        acc = acc + jnp.dot(xm, wv_ref[kk, :in_dim, :],
                            preferred_element_type=jnp.float32)
        acc = acc + jnp.dot(mask, wv_ref[kk, in_dim:, :],
                            preferred_element_type=jnp.float32)
    out_ref[:] = acc + bias_ref[:]


def kernel(x, slopes, intercepts, bias):
    b, in_dim = x.shape
    out_dim, _, k = slopes.shape
    s_t = jnp.transpose(slopes, (2, 1, 0))
    t_t = jnp.transpose(intercepts, (2, 1, 0))
    w = jnp.concatenate([s_t, t_t], axis=1).astype(jnp.bfloat16)
    bias2d = bias.reshape(1, out_dim)

    return pl.pallas_call(
        _spline_body,
        in_specs=[
            pl.BlockSpec(memory_space=pltpu.MemorySpace.VMEM),
            pl.BlockSpec(memory_space=pl.ANY),
            pl.BlockSpec(memory_space=pltpu.MemorySpace.VMEM),
        ],
        out_specs=pl.BlockSpec(memory_space=pltpu.MemorySpace.VMEM),
        out_shape=jax.ShapeDtypeStruct((b, out_dim), jnp.float32),
        scratch_shapes=[
            pltpu.VMEM((k, 2 * in_dim, out_dim), jnp.bfloat16),
            pltpu.SemaphoreType.DMA((k,)),
        ],
    )(x, w, bias2d)


# speedup vs baseline: 24.7583x; 1.4422x over previous
import jax
import jax.numpy as jnp
from jax.experimental import pallas as pl

INPUT_MIN, INPUT_MAX = 0.0, 1.0


def _spline_body(x_ref, w_ref, bias_ref, out_ref):
    num_k = w_ref.shape[0]
    in_dim = x_ref.shape[1]
    xv = x_ref[:]
    x_norm = (xv - INPUT_MIN) / (INPUT_MAX - INPUT_MIN)
    idx = jnp.clip(jnp.floor(x_norm * num_k), 0.0, num_k - 1.0).astype(jnp.bfloat16)
    xbf = xv.astype(jnp.bfloat16)
    acc = jnp.zeros((xv.shape[0], w_ref.shape[2]), jnp.float32)
    for kk in range(num_k):
        sel = idx == jnp.bfloat16(kk)
        xm = jnp.where(sel, xbf, jnp.bfloat16(0))
        mask = jnp.where(sel, jnp.bfloat16(1), jnp.bfloat16(0))
        acc = acc + jnp.dot(xm, w_ref[kk, :in_dim, :],
                            preferred_element_type=jnp.float32)
        acc = acc + jnp.dot(mask, w_ref[kk, in_dim:, :],
                            preferred_element_type=jnp.float32)
    out_ref[:] = acc + bias_ref[:]


def kernel(x, slopes, intercepts, bias):
    b, in_dim = x.shape
    out_dim, _, k = slopes.shape
    s_t = jnp.transpose(slopes, (2, 1, 0))
    t_t = jnp.transpose(intercepts, (2, 1, 0))
    w = jnp.concatenate([s_t, t_t], axis=1).astype(jnp.bfloat16)
    bias2d = bias.reshape(1, out_dim)

    return pl.pallas_call(
        _spline_body,
        out_shape=jax.ShapeDtypeStruct((b, out_dim), jnp.float32),
    )(x, w, bias2d)
